# Initial kernel scaffold; baseline (speedup 1.0000x reference)
import jax
import jax.numpy as jnp
from jax.experimental import pallas as pl


def kernel(positions, properties, W1, b1, W2, b2, W3, b3):
    raise NotImplementedError("write your pallas kernel here")


# speedup vs baseline: 5.2628x; 5.2628x over previous
import jax
import jax.numpy as jnp
from jax.experimental import pallas as pl
from jax.experimental.pallas import tpu as pltpu

_N = 8192
_R = 256
_K = 5


def _rne_bf16(x):
    u = jax.lax.bitcast_convert_type(x, jnp.uint32)
    r = (u + 0x7FFF + ((u >> 16) & 1)) & jnp.uint32(0xFFFF0000)
    return jax.lax.bitcast_convert_type(r, jnp.float32)


def _knn_block(t_ref, pt_ref, p2_ref, props_ref, out_ref):
    tx = t_ref[:, 0:1]
    ty = t_ref[:, 1:2]
    px = pt_ref[0:1, :]
    py = pt_ref[1:2, :]
    t2 = tx * tx + ty * ty
    txb = _rne_bf16(tx)
    tyb = _rne_bf16(ty)
    pxb = _rne_bf16(px)
    pyb = _rne_bf16(py)
    ab = txb * pxb + tyb * pyb
    d2 = (t2 + p2_ref[0:1, :]) - 2.0 * ab
    d2 = jnp.maximum(d2, 1e-12)

    iota = jax.lax.broadcasted_iota(jnp.int32, (_R, _N), 1)
    wmat = jnp.zeros((_R, _N), jnp.float32)
    wsum = jnp.zeros((_R, 1), jnp.float32)
    for k in range(_K):
        m = jnp.min(d2, axis=1, keepdims=True)
        cand = jnp.where(d2 == m, iota, jnp.int32(_N))
        idx = jnp.min(cand, axis=1, keepdims=True)
        wk = 1.0 / (jnp.sqrt(m) + 1e-8)
        mask = iota == idx
        wmat = jnp.where(mask, wk, wmat)
        wsum = wsum + wk
        if k < _K - 1:
            d2 = jnp.where(mask, jnp.float32(jnp.inf), d2)

    acc = jnp.dot(wmat, props_ref[...], precision=jax.lax.Precision.HIGHEST,
                  preferred_element_type=jnp.float32)
    out_ref[...] = acc * (1.0 / wsum)


def kernel(positions, properties, W1, b1, W2, b2, W3, b3):
    n = positions.shape[0]
    h = jax.nn.relu(positions @ W1.T + b1)
    h = jax.nn.relu(h @ W2.T + b2)
    theta = (h @ W3.T + b3).reshape(n, 2, 3)
    pos_h = jnp.concatenate([positions, jnp.ones_like(positions[:, :1])], axis=1)
    transformed = jnp.einsum('nij,nj->ni', theta, pos_h)

    pos_t = positions.T
    p2 = jnp.sum(positions * positions, axis=1, keepdims=True).T

    out = pl.pallas_call(
        _knn_block,
        grid=(n // _R,),
        in_specs=[
            pl.BlockSpec((_R, 2), lambda i: (i, 0)),
            pl.BlockSpec((2, _N), lambda i: (0, 0)),
            pl.BlockSpec((1, _N), lambda i: (0, 0)),
            pl.BlockSpec((_N, 3), lambda i: (0, 0)),
        ],
        out_specs=pl.BlockSpec((_R, 3), lambda i: (i, 0)),
        out_shape=jax.ShapeDtypeStruct((n, 3), jnp.float32),
        compiler_params=pltpu.CompilerParams(
            dimension_semantics=("parallel",)),
    )(transformed, pos_t, p2, properties)
    return out
